# baseline (speedup 1.0000x reference)
import jax
import jax.numpy as jnp
from jax.experimental import pallas as pl

N = 10000
NS = 16384
B = 256
OUT = 64


def _bn(x, g, b):
    m = jnp.mean(x, axis=0)
    v = jnp.var(x, axis=0)
    return (x - m) / jnp.sqrt(v + 1e-5) * g + b


def _segmean(x, idx, n):
    s = jax.ops.segment_sum(x, idx, num_segments=n)
    c = jax.ops.segment_sum(jnp.ones((x.shape[0], 1), x.dtype), idx, num_segments=n)
    return s / jnp.maximum(c, 1.0)


def _gcn(x, ei, ew, W, b, n):
    xw = x @ W
    row = jnp.concatenate([ei[0], jnp.arange(n)])
    col = jnp.concatenate([ei[1], jnp.arange(n)])
    w = jnp.concatenate([ew, jnp.ones((n,), xw.dtype)])
    deg = jax.ops.segment_sum(w, col, num_segments=n)
    dinv = jnp.where(deg > 0, 1.0 / jnp.sqrt(jnp.maximum(deg, 1e-12)), 0.0)
    norm = dinv[row] * w * dinv[col]
    out = jax.ops.segment_sum(norm[:, None] * xw[row], col, num_segments=n)
    return out + b


def _copy_kernel(x_ref, o_ref):
    o_ref[...] = x_ref[...]


def kernel(x, x_service, edge_attr_service, params, edge_index, edge_index_service, batch):
    with jax.default_matmul_precision("highest"):
        return _net(x, x_service, edge_attr_service, params, edge_index, edge_index_service, batch)


def _net(x, x_service, edge_attr_service, params, edge_index, edge_index_service, batch):
    idx = x[:, 0].astype(jnp.int32)
    h = jnp.concatenate([params["node_emb"][idx], x[:, 1:]], axis=-1)
    row, col = edge_index[0], edge_index[1]
    for lp in params["gin"]:
        agg = jax.ops.segment_sum(h[row], col, num_segments=N)
        z = (1.0 + lp["eps"]) * h + agg
        z = z @ lp["W1"] + lp["b1"]
        z = _bn(z, lp["g1"], lp["bt1"])
        z = jax.nn.relu(z)
        z = z @ lp["W2"] + lp["b2"]
        h = jax.nn.relu(_bn(z, lp["g"], lp["bt"]))
    sidx = x_service[:, 0].astype(jnp.int32)
    hs = jnp.concatenate([params["svc_emb"][sidx], x_service[:, 1:]], axis=-1)
    for lp in params["gcn"]:
        hs = _gcn(hs, edge_index_service, edge_attr_service, lp["W"], lp["b"], NS)
        hs = jax.nn.relu(_bn(hs, lp["g"], lp["bt"]))
    hs = hs @ params["svcLin_W"] + params["svcLin_b"]
    h = h @ params["nodeLin_W"] + params["nodeLin_b"]
    xg = _segmean(h, batch, B)
    service_batch = jnp.tile(jnp.arange(OUT), B)
    xs = _segmean(hs, service_batch, OUT)
    out = jax.nn.sigmoid(xg @ xs.T)
    out = pl.pallas_call(
        _copy_kernel,
        out_shape=jax.ShapeDtypeStruct(out.shape, out.dtype),
    )(out)
    return out


# speedup vs baseline: 3.7811x; 3.7811x over previous
import functools

import jax
import jax.numpy as jnp
from jax import lax
from jax.experimental import pallas as pl
from jax.experimental.pallas import tpu as pltpu
from jax.experimental.pallas import tpu_sc as plsc

N = 10000
E = 320000
NS = 16384
ES = 262144
B = 256
OUT = 64
H = 128
EMB = 128

F32 = jnp.float32
HIGH = jax.lax.Precision.HIGHEST

_NC = 2
_NSUB = 16

_GIN_K = 80
_GIN_NCH = 125
_N_PAD = 10240
_GCN_K = 128
_GCN_NCH = 128
_NHALF = NS // 2
_TRASH = 512
_DEG_PER_W = ES // (_NC * _NSUB)


def _zero_vmem_2d(buf, rows, cols):
    z = jnp.zeros((16,), F32)

    def body(r, _):
        for j in range(cols // 16):
            buf[r, pl.ds(j * 16, 16)] = z
        return 0

    lax.fori_loop(0, rows, body, 0)


_GIN_NHALF = _N_PAD // 2


def _gin_agg_body(h_hbm, row_hbm, col_hbm, out_hbm, idxr, idxc, idxc2, buf,
                  acc, sem):
    n_per_tile = _GIN_NHALF // _NSUB
    c = lax.axis_index("c")
    s = lax.axis_index("s")
    w = c * _NSUB + s
    pltpu.sync_copy(row_hbm.at[w], idxr)
    pltpu.sync_copy(col_hbm.at[w], idxc)
    lanes = lax.iota(jnp.int32, 16)
    for k in range(2):
        nbase = k * _GIN_NHALF
        _zero_vmem_2d(buf, _GIN_K, 128)
        base = s * n_per_tile
        for i in range(n_per_tile // _GIN_K):
            pltpu.sync_copy(buf, acc.at[pl.ds(base + i * _GIN_K, _GIN_K)])
        pltpu.sync_copy(buf.at[pl.ds(0, _TRASH // _NSUB)],
                        acc.at[pl.ds(_GIN_NHALF + s * (_TRASH // _NSUB),
                                     _TRASH // _NSUB)])
        plsc.subcore_barrier()

        def chunk(j, _):
            pltpu.async_copy(h_hbm.at[idxr.at[j]], buf, sem).wait()

            def remap(g, _):
                col16 = idxc[j, pl.ds(g * 16, 16)]
                t = col16 - nbase
                ok = (t >= 0) & (t < _GIN_NHALF)
                trash = _GIN_NHALF + ((j * 16 + g * 16 + lanes) & (_TRASH - 1))
                idxc2[pl.ds(g * 16, 16)] = jnp.where(ok, t, trash)
                return 0

            lax.fori_loop(0, _GIN_K // 16, remap, 0)
            pltpu.sync_copy(buf, acc.at[idxc2], add=True)
            return 0

        lax.fori_loop(0, _GIN_NCH, chunk, 0)
        plsc.subcore_barrier()
        pltpu.sync_copy(acc.at[pl.ds(s * n_per_tile, n_per_tile)],
                        out_hbm.at[pl.ds(c * _N_PAD + nbase + s * n_per_tile,
                                         n_per_tile)])
        plsc.subcore_barrier()


_gin_agg_kernel = pl.kernel(
    _gin_agg_body,
    out_type=jax.ShapeDtypeStruct((_NC * _N_PAD, 128), F32),
    mesh=plsc.VectorSubcoreMesh(core_axis_name="c", subcore_axis_name="s"),
    compiler_params=pltpu.CompilerParams(needs_layout_passes=False),
    scratch_types=[
        pltpu.VMEM((_GIN_NCH, _GIN_K), jnp.int32),
        pltpu.VMEM((_GIN_NCH, _GIN_K), jnp.int32),
        pltpu.VMEM((_GIN_K,), jnp.int32),
        pltpu.VMEM((_GIN_K, 128), F32),
        pltpu.VMEM_SHARED((_GIN_NHALF + _TRASH, 128), F32),
        pltpu.SemaphoreType.DMA,
    ],
)


def _deg_body(col_hbm, w_hbm, out_hbm, colv, wv, acc):
    c = lax.axis_index("c")
    s = lax.axis_index("s")
    w = c * _NSUB + s
    pltpu.sync_copy(col_hbm.at[pl.ds(w * _DEG_PER_W, _DEG_PER_W)], colv)
    pltpu.sync_copy(w_hbm.at[pl.ds(w * _DEG_PER_W, _DEG_PER_W)], wv)
    z = jnp.zeros((16,), F32)

    def zbody(i, _):
        acc[pl.ds(i * 16, 16)] = z
        return 0

    lax.fori_loop(0, NS // 16, zbody, 0)

    def ebody(i, _):
        idx = colv[pl.ds(i * 16, 16)]
        val = wv[pl.ds(i * 16, 16)]
        plsc.addupdate_scatter(acc, [idx], val)
        return 0

    lax.fori_loop(0, _DEG_PER_W // 16, ebody, 0)
    pltpu.sync_copy(acc, out_hbm.at[w])


_deg_kernel = pl.kernel(
    _deg_body,
    out_type=jax.ShapeDtypeStruct((_NC * _NSUB, NS), F32),
    mesh=plsc.VectorSubcoreMesh(core_axis_name="c", subcore_axis_name="s"),
    compiler_params=pltpu.CompilerParams(needs_layout_passes=False),
    scratch_types=[
        pltpu.VMEM((_DEG_PER_W,), jnp.int32),
        pltpu.VMEM((_DEG_PER_W,), F32),
        pltpu.VMEM((NS,), F32),
    ],
)


def _gcn_agg_body(y_hbm, ecat_hbm, out_hbm,
                  idxr, idxc, idxc2, wv, buf, acc, sem):
    c = lax.axis_index("c")
    s = lax.axis_index("s")
    nrows = _NHALF // _NSUB
    ne = ES // _GCN_K
    pltpu.sync_copy(ecat_hbm.at[pl.ds(s * _GCN_NCH, _GCN_NCH)], idxr)
    pltpu.sync_copy(ecat_hbm.at[pl.ds(ne + s * _GCN_NCH, _GCN_NCH)], idxc)
    off = jnp.full((16,), c * NS, jnp.int32)

    def roff(r, _):
        for g in range(_GCN_K // 16):
            sl = pl.ds(g * 16, 16)
            idxr[r, sl] = idxr[r, sl] + off
        return 0

    lax.fori_loop(0, _GCN_NCH, roff, 0)
    lanes = lax.iota(jnp.int32, 16)
    for k in range(2):
        nbase = k * _NHALF
        _zero_vmem_2d(buf, _GCN_K, 128)
        for i in range(nrows // _GCN_K):
            pltpu.sync_copy(buf, acc.at[pl.ds(s * nrows + i * _GCN_K, _GCN_K)])
        pltpu.sync_copy(buf.at[pl.ds(0, _TRASH // _NSUB)],
                        acc.at[pl.ds(_NHALF + s * (_TRASH // _NSUB),
                                     _TRASH // _NSUB)])
        plsc.subcore_barrier()

        def chunk(j, _):
            cp = pltpu.async_copy(y_hbm.at[idxr.at[j]], buf, sem)
            pltpu.sync_copy(ecat_hbm.at[2 * ne + s * _GCN_NCH + j], wv)
            cp.wait()

            def remap(g, _):
                col16 = idxc[j, pl.ds(g * 16, 16)]
                t = col16 - nbase
                ok = (t >= 0) & (t < _NHALF)
                trash = _NHALF + ((j * 16 + g * 16 + lanes) & (_TRASH - 1))
                idxc2[pl.ds(g * 16, 16)] = jnp.where(ok, t, trash)
                return 0

            lax.fori_loop(0, _GCN_K // 16, remap, 0)

            def scale(e4, _):
                for t in range(4):
                    e = e4 * 4 + t
                    sp = plsc.bitcast(plsc.load_gather(
                        wv, [jnp.full((16,), e, jnp.int32)]), F32)
                    for v in range(8):
                        buf[e, pl.ds(v * 16, 16)] = buf[e, pl.ds(v * 16, 16)] * sp
                return 0

            lax.fori_loop(0, _GCN_K // 4, scale, 0)
            pltpu.sync_copy(buf, acc.at[idxc2], add=True)
            return 0

        lax.fori_loop(0, _GCN_NCH, chunk, 0)
        plsc.subcore_barrier()
        pltpu.sync_copy(acc.at[pl.ds(s * nrows, nrows)],
                        out_hbm.at[pl.ds(c * NS + nbase + s * nrows, nrows)])
        plsc.subcore_barrier()


_gcn_agg_kernel = pl.kernel(
    _gcn_agg_body,
    out_type=jax.ShapeDtypeStruct((_NC * NS, 128), F32),
    mesh=plsc.VectorSubcoreMesh(core_axis_name="c", subcore_axis_name="s"),
    compiler_params=pltpu.CompilerParams(needs_layout_passes=False),
    scratch_types=[
        pltpu.VMEM((_GCN_NCH, _GCN_K), jnp.int32),
        pltpu.VMEM((_GCN_NCH, _GCN_K), jnp.int32),
        pltpu.VMEM((_GCN_K,), jnp.int32),
        pltpu.VMEM((_GCN_K,), jnp.int32),
        pltpu.VMEM((_GCN_K, 128), F32),
        pltpu.VMEM_SHARED((_NHALF + _TRASH, 128), F32),
        pltpu.SemaphoreType.DMA,
    ],
)


def _embed_body(nfeat, x_ref, emb_ref, o_ref):
    x = x_ref[...]
    idx = x[:, 0:1].astype(jnp.int32)
    oh = (idx == lax.broadcasted_iota(jnp.int32, (x.shape[0], 128), 1)).astype(F32)
    e = jnp.dot(oh, emb_ref[...], precision=HIGH, preferred_element_type=F32)
    pad = 144 - 128 - (nfeat - 1)
    o_ref[...] = jnp.concatenate(
        [e, x[:, 1:nfeat], jnp.zeros((x.shape[0], pad), F32)], axis=1)


def _embed(x, emb_pad, nfeat):
    n = x.shape[0]
    nb = 10 if n == N else 8
    blk = n // nb
    return pl.pallas_call(
        functools.partial(_embed_body, nfeat),
        grid=(nb,),
        in_specs=[
            pl.BlockSpec((blk, x.shape[1]), lambda i: (i, 0)),
            pl.BlockSpec((128, 128), lambda i: (0, 0)),
        ],
        out_specs=pl.BlockSpec((blk, 144), lambda i: (i, 0)),
        out_shape=jax.ShapeDtypeStruct((n, 144), F32),
    )(x, emb_pad)


def _gin_comb_body(pa1_ref, pa2_ref, pc1_ref, pc2_ref, o_ref):
    a = pa1_ref[...] + pa2_ref[...]
    c = pc1_ref[...] + pc2_ref[...]
    o_ref[...] = jnp.concatenate([a, c[:, 112:128]], axis=1)


def _gin_comb(pa, pc):
    blk = _N_PAD // 8
    return pl.pallas_call(
        _gin_comb_body,
        grid=(8,),
        in_specs=[
            pl.BlockSpec((blk, 128), lambda i: (i, 0)),
            pl.BlockSpec((blk, 128), lambda i: (i + 8, 0)),
            pl.BlockSpec((blk, 128), lambda i: (i, 0)),
            pl.BlockSpec((blk, 128), lambda i: (i + 8, 0)),
        ],
        out_specs=pl.BlockSpec((blk, 144), lambda i: (i, 0)),
        out_shape=jax.ShapeDtypeStruct((_N_PAD, 144), F32),
    )(pa, pa, pc, pc)


def _gin_dense1_body(h_ref, agg_ref, eps_ref, w1_ref, b1_ref, g1_ref,
                     bt1_ref, w2_ref, b2_ref, g_ref, bt_ref, o_ref):
    n = h_ref.shape[0]
    z = (1.0 + eps_ref[0, 0]) * h_ref[...] + agg_ref[pl.ds(0, n)]
    _gin_mlp(z, w1_ref, b1_ref, g1_ref, bt1_ref, w2_ref, b2_ref, g_ref,
             bt_ref, o_ref)


def _gin_dense_body(h_ref, p_ref, eps_ref, w1_ref, b1_ref, g1_ref, bt1_ref,
                    w2_ref, b2_ref, g_ref, bt_ref, o_ref):
    n = h_ref.shape[0]
    z = (1.0 + eps_ref[0, 0]) * h_ref[...] + p_ref[pl.ds(0, n)] + p_ref[pl.ds(_N_PAD, n)]
    _gin_mlp(z, w1_ref, b1_ref, g1_ref, bt1_ref, w2_ref, b2_ref, g_ref,
             bt_ref, o_ref)


def _gin_mlp(z, w1_ref, b1_ref, g1_ref, bt1_ref, w2_ref, b2_ref, g_ref,
             bt_ref, o_ref):
    u = jnp.dot(z, w1_ref[...], precision=HIGH, preferred_element_type=F32) + b1_ref[...]
    m = jnp.mean(u, axis=0, keepdims=True)
    v = jnp.mean((u - m) ** 2, axis=0, keepdims=True)
    u = (u - m) / jnp.sqrt(v + 1e-5) * g1_ref[...] + bt1_ref[...]
    u = jnp.maximum(u, 0.0)
    u2 = jnp.dot(u, w2_ref[...], precision=HIGH, preferred_element_type=F32) + b2_ref[...]
    m2 = jnp.mean(u2, axis=0, keepdims=True)
    v2 = jnp.mean((u2 - m2) ** 2, axis=0, keepdims=True)
    u2 = (u2 - m2) / jnp.sqrt(v2 + 1e-5) * g_ref[...] + bt_ref[...]
    o_ref[...] = jnp.maximum(u2, 0.0)


def _gin_params(lp, w1p):
    return (jnp.reshape(lp["eps"], (1, 1)), w1p,
            jnp.reshape(lp["b1"], (1, -1)), jnp.reshape(lp["g1"], (1, -1)),
            jnp.reshape(lp["bt1"], (1, -1)), lp["W2"],
            jnp.reshape(lp["b2"], (1, -1)), jnp.reshape(lp["g"], (1, -1)),
            jnp.reshape(lp["bt"], (1, -1)))


def _gcn_pre1_body(hs_ref, w_ref, dinv_ref, o_ref):
    y = jnp.dot(hs_ref[...], w_ref[0], precision=HIGH, preferred_element_type=F32)
    o_ref[...] = (dinv_ref[...] * y)[None]


_RB = NS // 8


def _gcn_pre1(hs, w2, dinv):
    din = hs.shape[1]
    return pl.pallas_call(
        _gcn_pre1_body,
        grid=(2, 8),
        in_specs=[
            pl.BlockSpec((_RB, din), lambda q, i: (i, 0)),
            pl.BlockSpec((1, din, 128), lambda q, i: (q, 0, 0)),
            pl.BlockSpec((_RB, 1), lambda q, i: (i, 0)),
        ],
        out_specs=pl.BlockSpec((1, _RB, 128), lambda q, i: (q, i, 0)),
        out_shape=jax.ShapeDtypeStruct((2, NS, 128), F32),
    )(hs, w2, dinv)


def _gcn_pre2_body(hs_ref, w_ref, dinv_ref, o_ref):
    acc = jnp.dot(hs_ref[0], w_ref[0, 0], precision=HIGH, preferred_element_type=F32)
    acc = acc + jnp.dot(hs_ref[1], w_ref[0, 1], precision=HIGH,
                        preferred_element_type=F32)
    o_ref[...] = (dinv_ref[...] * acc)[None]


def _gcn_pre2(hs2, wqq, dinv):
    return pl.pallas_call(
        _gcn_pre2_body,
        grid=(2, 8),
        in_specs=[
            pl.BlockSpec((2, _RB, 128), lambda q, i: (0, i, 0)),
            pl.BlockSpec((1, 2, 128, 128), lambda q, i: (q, 0, 0, 0)),
            pl.BlockSpec((_RB, 1), lambda q, i: (i, 0)),
        ],
        out_specs=pl.BlockSpec((1, _RB, 128), lambda q, i: (q, i, 0)),
        out_shape=jax.ShapeDtypeStruct((2, NS, 128), F32),
    )(hs2, wqq, dinv)


def _gcn_stat_body(agg_ref, y_ref, dinv_ref, b_ref, u_ref, ps_ref, pss_ref):
    t = dinv_ref[...] * (agg_ref[0] + y_ref[0]) + b_ref[0]
    u_ref[...] = t[None]
    ps_ref[...] = jnp.sum(t, axis=0, keepdims=True)[None, None]
    pss_ref[...] = jnp.sum(t * t, axis=0, keepdims=True)[None, None]


def _gcn_scale_body(ps_ref, pss_ref, g_ref, bt_ref, sc_ref, sh_ref):
    m = jnp.sum(ps_ref[...], axis=1) / NS
    v = jnp.sum(pss_ref[...], axis=1) / NS - m * m
    sc = g_ref[...] / jnp.sqrt(v + 1e-5)
    sc_ref[...] = sc
    sh_ref[...] = bt_ref[...] - m * sc


def _gcn_app_body(u_ref, sc_ref, sh_ref, o_ref):
    o_ref[...] = jnp.maximum(u_ref[...] * sc_ref[...] + sh_ref[...], 0.0)


def _gcn_post(agg2, y2, dinv, lp):
    vec = pl.BlockSpec((1, 1, 128), lambda q, i: (q, 0, 0))
    big = pl.BlockSpec((1, _RB, 128), lambda q, i: (q, i, 0))
    part = pl.BlockSpec((1, 1, 1, 128), lambda q, i: (q, i, 0, 0))
    u, ps, pss = pl.pallas_call(
        _gcn_stat_body,
        grid=(2, 8),
        in_specs=[
            big, big,
            pl.BlockSpec((_RB, 1), lambda q, i: (i, 0)),
            vec,
        ],
        out_specs=[big, part, part],
        out_shape=[
            jax.ShapeDtypeStruct((2, NS, 128), F32),
            jax.ShapeDtypeStruct((2, 8, 1, 128), F32),
            jax.ShapeDtypeStruct((2, 8, 1, 128), F32),
        ],
    )(agg2, y2, dinv, jnp.reshape(lp["b"], (2, 1, 128)))
    sc, sh = pl.pallas_call(
        _gcn_scale_body,
        out_shape=[
            jax.ShapeDtypeStruct((2, 1, 128), F32),
            jax.ShapeDtypeStruct((2, 1, 128), F32),
        ],
    )(ps, pss, jnp.reshape(lp["g"], (2, 1, 128)),
      jnp.reshape(lp["bt"], (2, 1, 128)))
    return pl.pallas_call(
        _gcn_app_body,
        grid=(2, 8),
        in_specs=[big, vec, vec],
        out_specs=big,
        out_shape=jax.ShapeDtypeStruct((2, NS, 128), F32),
    )(u, sc, sh)


def _svc_pool_body(hs_ref, w_ref, b_ref, o_ref):
    i = pl.program_id(0)
    hl = jnp.dot(hs_ref[0], w_ref[0], precision=HIGH, preferred_element_type=F32)
    hl = hl + jnp.dot(hs_ref[1], w_ref[1], precision=HIGH,
                      preferred_element_type=F32)
    part = jnp.sum(jnp.reshape(hl, (_RB // OUT, OUT, H)), axis=0)

    @pl.when(i == 0)
    def _():
        o_ref[...] = part

    @pl.when(i > 0)
    def _():
        o_ref[...] = o_ref[...] + part

    @pl.when(i == 7)
    def _():
        o_ref[...] = o_ref[...] / B + b_ref[...]


def _svc_pool(hs2, svc_w2, b):
    return pl.pallas_call(
        _svc_pool_body,
        grid=(8,),
        in_specs=[
            pl.BlockSpec((2, _RB, 128), lambda i: (0, i, 0)),
            pl.BlockSpec((2, 128, H), lambda i: (0, 0, 0)),
            pl.BlockSpec((1, H), lambda i: (0, 0)),
        ],
        out_specs=pl.BlockSpec((OUT, H), lambda i: (0, 0)),
        out_shape=jax.ShapeDtypeStruct((OUT, H), F32),
    )(hs2, svc_w2, b)


_NPB = N // 10


def _node_pool_body(h_ref, w_ref, b_ref, batch_ref, o_ref, cnt_ref):
    i = pl.program_id(0)
    hl = jnp.dot(h_ref[...], w_ref[...], precision=HIGH,
                 preferred_element_type=F32) + b_ref[...]
    bt = batch_ref[0]
    oh = (lax.broadcasted_iota(jnp.int32, (B, _NPB), 0) == bt).astype(F32)
    s = jnp.dot(oh, hl, precision=HIGH, preferred_element_type=F32)
    cnt = jnp.sum(oh, axis=1, keepdims=True) + jnp.zeros((B, H), F32)

    @pl.when(i == 0)
    def _():
        o_ref[...] = s
        cnt_ref[...] = cnt

    @pl.when(i > 0)
    def _():
        o_ref[...] = o_ref[...] + s
        cnt_ref[...] = cnt_ref[...] + cnt

    @pl.when(i == 9)
    def _():
        o_ref[...] = o_ref[...] / jnp.maximum(cnt_ref[...], 1.0)


def _node_pool(h, w, b, batch3):
    return pl.pallas_call(
        _node_pool_body,
        grid=(10,),
        in_specs=[
            pl.BlockSpec((_NPB, H), lambda i: (i, 0)),
            pl.BlockSpec((H, H), lambda i: (0, 0)),
            pl.BlockSpec((1, H), lambda i: (0, 0)),
            pl.BlockSpec((1, 1, _NPB), lambda i: (i, 0, 0)),
        ],
        out_specs=pl.BlockSpec((B, H), lambda i: (0, 0)),
        out_shape=jax.ShapeDtypeStruct((B, H), F32),
        scratch_shapes=[pltpu.VMEM((B, H), F32)],
    )(h, w, b, batch3)


def _final_body(xg_ref, xs_ref, o_ref):
    logits = lax.dot_general(xg_ref[...], xs_ref[...], (((1,), (1,)), ((), ())),
                             precision=HIGH, preferred_element_type=F32)
    o_ref[...] = jax.nn.sigmoid(logits)


def _pad_rows(w, rows):
    return jnp.concatenate([w, jnp.zeros((rows - w.shape[0], w.shape[1]), w.dtype)], 0)


def kernel(x, x_service, edge_attr_service, params, edge_index, edge_index_service, batch):
    row = edge_index[0].astype(jnp.int32)
    col = edge_index[1].astype(jnp.int32)
    row3 = jnp.reshape(row, (_NC * _NSUB, _GIN_NCH, _GIN_K))
    col3 = jnp.reshape(col, (_NC * _NSUB, _GIN_NCH, _GIN_K))
    rows_s = edge_index_service[0].astype(jnp.int32)
    cols_s = edge_index_service[1].astype(jnp.int32)
    ecat = jnp.concatenate(
        [jnp.reshape(rows_s, (ES // _GCN_K, _GCN_K)),
         jnp.reshape(cols_s, (ES // _GCN_K, _GCN_K)),
         jax.lax.bitcast_convert_type(
             jnp.reshape(edge_attr_service, (ES // _GCN_K, _GCN_K)), jnp.int32)],
        axis=0)
    emb_n = _pad_rows(params["node_emb"], 128)
    emb_s = _pad_rows(params["svc_emb"], 128)
    gin_w1 = [_pad_rows(params["gin"][0]["W1"], 144),
              params["gin"][1]["W1"], params["gin"][2]["W1"]]
    gcn_w1 = jnp.transpose(
        jnp.reshape(_pad_rows(params["gcn"][0]["W"], 144), (144, 2, 128)), (1, 0, 2))
    gcn_w2 = jnp.transpose(
        jnp.reshape(params["gcn"][1]["W"], (2, 128, 2, 128)), (2, 0, 1, 3))
    svc_w2 = jnp.reshape(params["svcLin_W"], (2, 128, H))
    batch3 = jnp.reshape(batch.astype(jnp.int32), (10, 1, _NPB))

    h = _embed(x, emb_n, 7)
    lp = params["gin"][0]
    pa = _gin_agg_kernel(h[:, 0:128], row3, col3)
    pc = _gin_agg_kernel(h[:, 16:144], row3, col3)
    agg1 = _gin_comb(pa, pc)
    h = pl.pallas_call(
        _gin_dense1_body,
        out_shape=jax.ShapeDtypeStruct((N, H), F32),
    )(h, agg1, *_gin_params(lp, gin_w1[0]))
    for i in (1, 2):
        p = _gin_agg_kernel(h, row3, col3)
        h = pl.pallas_call(
            _gin_dense_body,
            out_shape=jax.ShapeDtypeStruct((N, H), F32),
        )(h, p, *_gin_params(params["gin"][i], gin_w1[i]))

    hs = _embed(x_service, emb_s, 5)
    degp = _deg_kernel(cols_s, edge_attr_service)
    deg = jnp.sum(degp, axis=0) + 1.0
    dinv = jnp.reshape(1.0 / jnp.sqrt(deg), (NS, 1))
    y2 = _gcn_pre1(hs, gcn_w1, dinv)
    agg2 = _gcn_agg_kernel(jnp.reshape(y2, (2 * NS, 128)), ecat)
    hs2 = _gcn_post(jnp.reshape(agg2, (2, NS, 128)), y2, dinv, params["gcn"][0])
    y2 = _gcn_pre2(hs2, gcn_w2, dinv)
    agg2 = _gcn_agg_kernel(jnp.reshape(y2, (2 * NS, 128)), ecat)
    hs2 = _gcn_post(jnp.reshape(agg2, (2, NS, 128)), y2, dinv, params["gcn"][1])

    xs = _svc_pool(hs2, svc_w2, jnp.reshape(params["svcLin_b"], (1, -1)))
    xg = _node_pool(h, params["nodeLin_W"],
                    jnp.reshape(params["nodeLin_b"], (1, -1)), batch3)
    return pl.pallas_call(
        _final_body,
        out_shape=jax.ShapeDtypeStruct((B, OUT), F32),
    )(xg, xs)


# speedup vs baseline: 4.8831x; 1.2914x over previous
import functools

import jax
import jax.numpy as jnp
from jax import lax
from jax.experimental import pallas as pl
from jax.experimental.pallas import tpu as pltpu
from jax.experimental.pallas import tpu_sc as plsc

N = 10000
E = 320000
NS = 16384
ES = 262144
B = 256
OUT = 64
H = 128
EMB = 128

F32 = jnp.float32
HIGH = jax.lax.Precision.HIGHEST

_NC = 2
_NSUB = 16

_GIN_K = 80
_GIN_NCH = 125
_N_PAD = 10240
_GCN_K = 128
_GCN_NCH = 128
_NHALF = NS // 2
_TRASH = 512
_DEG_PER_W = ES // (_NC * _NSUB)


def _zero_vmem_2d(buf, rows, cols):
    z = jnp.zeros((16,), F32)

    def body(r, _):
        for j in range(cols // 16):
            buf[r, pl.ds(j * 16, 16)] = z
        return 0

    lax.fori_loop(0, rows, body, 0)


_GIN_NHALF = _N_PAD // 2


def _gin_agg_body(h_hbm, row_hbm, col_hbm, out_hbm, idxr, idxc, buf, acc, sem):
    n_per_tile = _N_PAD // _NSUB
    c = lax.axis_index("c")
    s = lax.axis_index("s")
    w = c * _NSUB + s
    pltpu.sync_copy(row_hbm.at[w], idxr)
    pltpu.sync_copy(col_hbm.at[w], idxc)
    _zero_vmem_2d(buf, _GIN_K, 128)
    base = s * n_per_tile
    for i in range(n_per_tile // _GIN_K):
        pltpu.sync_copy(buf, acc.at[pl.ds(base + i * _GIN_K, _GIN_K)])
    plsc.subcore_barrier()

    def chunk(j, _):
        pltpu.async_copy(h_hbm.at[idxr.at[j]], buf, sem).wait()
        pltpu.sync_copy(buf, acc.at[idxc.at[j]], add=True)
        return 0

    lax.fori_loop(0, _GIN_NCH, chunk, 0)
    plsc.subcore_barrier()
    pltpu.sync_copy(acc.at[pl.ds(base, n_per_tile)],
                    out_hbm.at[pl.ds(c * _N_PAD + base, n_per_tile)])


_gin_agg_kernel = pl.kernel(
    _gin_agg_body,
    out_type=jax.ShapeDtypeStruct((_NC * _N_PAD, 128), F32),
    mesh=plsc.VectorSubcoreMesh(core_axis_name="c", subcore_axis_name="s"),
    compiler_params=pltpu.CompilerParams(needs_layout_passes=False),
    scratch_types=[
        pltpu.VMEM((_GIN_NCH, _GIN_K), jnp.int32),
        pltpu.VMEM((_GIN_NCH, _GIN_K), jnp.int32),
        pltpu.VMEM((_GIN_K, 128), F32),
        pltpu.VMEM_SHARED((_N_PAD, 128), F32),
        pltpu.SemaphoreType.DMA,
    ],
)


def _deg_body(col_hbm, w_hbm, out_hbm, colv, wv, acc):
    c = lax.axis_index("c")
    s = lax.axis_index("s")
    w = c * _NSUB + s
    pltpu.sync_copy(col_hbm.at[pl.ds(w * _DEG_PER_W, _DEG_PER_W)], colv)
    pltpu.sync_copy(w_hbm.at[pl.ds(w * _DEG_PER_W, _DEG_PER_W)], wv)
    z = jnp.zeros((16,), F32)

    def zbody(i, _):
        acc[pl.ds(i * 16, 16)] = z
        return 0

    lax.fori_loop(0, NS // 16, zbody, 0)

    def ebody(i, _):
        idx = colv[pl.ds(i * 16, 16)]
        val = wv[pl.ds(i * 16, 16)]
        plsc.addupdate_scatter(acc, [idx], val)
        return 0

    lax.fori_loop(0, _DEG_PER_W // 16, ebody, 0)
    pltpu.sync_copy(acc, out_hbm.at[w])


_deg_kernel = pl.kernel(
    _deg_body,
    out_type=jax.ShapeDtypeStruct((_NC * _NSUB, NS), F32),
    mesh=plsc.VectorSubcoreMesh(core_axis_name="c", subcore_axis_name="s"),
    compiler_params=pltpu.CompilerParams(needs_layout_passes=False),
    scratch_types=[
        pltpu.VMEM((_DEG_PER_W,), jnp.int32),
        pltpu.VMEM((_DEG_PER_W,), F32),
        pltpu.VMEM((NS,), F32),
    ],
)


def _gcn_agg_body(y_hbm, ecat_hbm, out_hbm,
                  idxr, idxc, idxc2, wv, buf, acc, sem):
    c = lax.axis_index("c")
    s = lax.axis_index("s")
    nrows = _NHALF // _NSUB
    ne = ES // _GCN_K
    pltpu.sync_copy(ecat_hbm.at[pl.ds(s * _GCN_NCH, _GCN_NCH)], idxr)
    pltpu.sync_copy(ecat_hbm.at[pl.ds(ne + s * _GCN_NCH, _GCN_NCH)], idxc)
    off = jnp.full((16,), c * NS, jnp.int32)

    def roff(r, _):
        for g in range(_GCN_K // 16):
            sl = pl.ds(g * 16, 16)
            idxr[r, sl] = idxr[r, sl] + off
        return 0

    lax.fori_loop(0, _GCN_NCH, roff, 0)
    lanes = lax.iota(jnp.int32, 16)
    for k in range(2):
        nbase = k * _NHALF
        _zero_vmem_2d(buf, _GCN_K, 128)
        for i in range(nrows // _GCN_K):
            pltpu.sync_copy(buf, acc.at[pl.ds(s * nrows + i * _GCN_K, _GCN_K)])
        pltpu.sync_copy(buf.at[pl.ds(0, _TRASH // _NSUB)],
                        acc.at[pl.ds(_NHALF + s * (_TRASH // _NSUB),
                                     _TRASH // _NSUB)])
        plsc.subcore_barrier()

        def chunk(j, _):
            cp = pltpu.async_copy(y_hbm.at[idxr.at[j]], buf, sem)
            pltpu.sync_copy(ecat_hbm.at[2 * ne + s * _GCN_NCH + j], wv)
            cp.wait()

            def remap(g, _):
                col16 = idxc[j, pl.ds(g * 16, 16)]
                t = col16 - nbase
                ok = (t >= 0) & (t < _NHALF)
                trash = _NHALF + ((j * 16 + g * 16 + lanes) & (_TRASH - 1))
                idxc2[pl.ds(g * 16, 16)] = jnp.where(ok, t, trash)
                return 0

            lax.fori_loop(0, _GCN_K // 16, remap, 0)

            def scale(e4, _):
                for t in range(4):
                    e = e4 * 4 + t
                    sp = plsc.bitcast(plsc.load_gather(
                        wv, [jnp.full((16,), e, jnp.int32)]), F32)
                    for v in range(8):
                        buf[e, pl.ds(v * 16, 16)] = buf[e, pl.ds(v * 16, 16)] * sp
                return 0

            lax.fori_loop(0, _GCN_K // 4, scale, 0)
            pltpu.sync_copy(buf, acc.at[idxc2], add=True)
            return 0

        lax.fori_loop(0, _GCN_NCH, chunk, 0)
        plsc.subcore_barrier()
        pltpu.sync_copy(acc.at[pl.ds(s * nrows, nrows)],
                        out_hbm.at[pl.ds(c * NS + nbase + s * nrows, nrows)])
        plsc.subcore_barrier()


_gcn_agg_kernel = pl.kernel(
    _gcn_agg_body,
    out_type=jax.ShapeDtypeStruct((_NC * NS, 128), F32),
    mesh=plsc.VectorSubcoreMesh(core_axis_name="c", subcore_axis_name="s"),
    compiler_params=pltpu.CompilerParams(needs_layout_passes=False),
    scratch_types=[
        pltpu.VMEM((_GCN_NCH, _GCN_K), jnp.int32),
        pltpu.VMEM((_GCN_NCH, _GCN_K), jnp.int32),
        pltpu.VMEM((_GCN_K,), jnp.int32),
        pltpu.VMEM((_GCN_K,), jnp.int32),
        pltpu.VMEM((_GCN_K, 128), F32),
        pltpu.VMEM_SHARED((_NHALF + _TRASH, 128), F32),
        pltpu.SemaphoreType.DMA,
    ],
)


def _embed_body(nfeat, x_ref, emb_ref, o_ref):
    x = x_ref[...]
    idx = x[:, 0:1].astype(jnp.int32)
    oh = (idx == lax.broadcasted_iota(jnp.int32, (x.shape[0], 128), 1)).astype(F32)
    e = jnp.dot(oh, emb_ref[...], precision=HIGH, preferred_element_type=F32)
    pad = 144 - 128 - (nfeat - 1)
    o_ref[...] = jnp.concatenate(
        [e, x[:, 1:nfeat], jnp.zeros((x.shape[0], pad), F32)], axis=1)


def _embed(x, emb_pad, nfeat):
    n = x.shape[0]
    nb = 10 if n == N else 8
    blk = n // nb
    return pl.pallas_call(
        functools.partial(_embed_body, nfeat),
        grid=(nb,),
        in_specs=[
            pl.BlockSpec((blk, x.shape[1]), lambda i: (i, 0)),
            pl.BlockSpec((128, 128), lambda i: (0, 0)),
        ],
        out_specs=pl.BlockSpec((blk, 144), lambda i: (i, 0)),
        out_shape=jax.ShapeDtypeStruct((n, 144), F32),
    )(x, emb_pad)


def _gin_comb_body(pa1_ref, pa2_ref, pc1_ref, pc2_ref, o_ref):
    a = pa1_ref[...] + pa2_ref[...]
    c = pc1_ref[...] + pc2_ref[...]
    o_ref[...] = jnp.concatenate([a, c[:, 112:128]], axis=1)


def _gin_comb(pa, pc):
    blk = _N_PAD // 8
    return pl.pallas_call(
        _gin_comb_body,
        grid=(8,),
        in_specs=[
            pl.BlockSpec((blk, 128), lambda i: (i, 0)),
            pl.BlockSpec((blk, 128), lambda i: (i + 8, 0)),
            pl.BlockSpec((blk, 128), lambda i: (i, 0)),
            pl.BlockSpec((blk, 128), lambda i: (i + 8, 0)),
        ],
        out_specs=pl.BlockSpec((blk, 144), lambda i: (i, 0)),
        out_shape=jax.ShapeDtypeStruct((_N_PAD, 144), F32),
    )(pa, pa, pc, pc)


def _gin_dense1_body(h_ref, agg_ref, eps_ref, w1_ref, b1_ref, g1_ref,
                     bt1_ref, w2_ref, b2_ref, g_ref, bt_ref, o_ref):
    n = h_ref.shape[0]
    z = (1.0 + eps_ref[0, 0]) * h_ref[...] + agg_ref[pl.ds(0, n)]
    _gin_mlp(z, w1_ref, b1_ref, g1_ref, bt1_ref, w2_ref, b2_ref, g_ref,
             bt_ref, o_ref)


def _gin_dense_body(h_ref, p_ref, eps_ref, w1_ref, b1_ref, g1_ref, bt1_ref,
                    w2_ref, b2_ref, g_ref, bt_ref, o_ref):
    n = h_ref.shape[0]
    z = (1.0 + eps_ref[0, 0]) * h_ref[...] + p_ref[pl.ds(0, n)] + p_ref[pl.ds(_N_PAD, n)]
    _gin_mlp(z, w1_ref, b1_ref, g1_ref, bt1_ref, w2_ref, b2_ref, g_ref,
             bt_ref, o_ref)


def _gin_mlp(z, w1_ref, b1_ref, g1_ref, bt1_ref, w2_ref, b2_ref, g_ref,
             bt_ref, o_ref):
    u = jnp.dot(z, w1_ref[...], precision=HIGH, preferred_element_type=F32) + b1_ref[...]
    m = jnp.mean(u, axis=0, keepdims=True)
    v = jnp.mean((u - m) ** 2, axis=0, keepdims=True)
    u = (u - m) / jnp.sqrt(v + 1e-5) * g1_ref[...] + bt1_ref[...]
    u = jnp.maximum(u, 0.0)
    u2 = jnp.dot(u, w2_ref[...], precision=HIGH, preferred_element_type=F32) + b2_ref[...]
    m2 = jnp.mean(u2, axis=0, keepdims=True)
    v2 = jnp.mean((u2 - m2) ** 2, axis=0, keepdims=True)
    u2 = (u2 - m2) / jnp.sqrt(v2 + 1e-5) * g_ref[...] + bt_ref[...]
    o_ref[...] = jnp.maximum(u2, 0.0)


def _gin_params(lp, w1p):
    return (jnp.reshape(lp["eps"], (1, 1)), w1p,
            jnp.reshape(lp["b1"], (1, -1)), jnp.reshape(lp["g1"], (1, -1)),
            jnp.reshape(lp["bt1"], (1, -1)), lp["W2"],
            jnp.reshape(lp["b2"], (1, -1)), jnp.reshape(lp["g"], (1, -1)),
            jnp.reshape(lp["bt"], (1, -1)))


def _gcn_pre1_body(hs_ref, w_ref, dinv_ref, o_ref):
    y = jnp.dot(hs_ref[...], w_ref[0], precision=HIGH, preferred_element_type=F32)
    o_ref[...] = (dinv_ref[...] * y)[None]


_RB = NS // 8


def _gcn_pre1(hs, w2, dinv):
    din = hs.shape[1]
    return pl.pallas_call(
        _gcn_pre1_body,
        grid=(2, 8),
        in_specs=[
            pl.BlockSpec((_RB, din), lambda q, i: (i, 0)),
            pl.BlockSpec((1, din, 128), lambda q, i: (q, 0, 0)),
            pl.BlockSpec((_RB, 1), lambda q, i: (i, 0)),
        ],
        out_specs=pl.BlockSpec((1, _RB, 128), lambda q, i: (q, i, 0)),
        out_shape=jax.ShapeDtypeStruct((2, NS, 128), F32),
    )(hs, w2, dinv)


def _gcn_pre2_body(hs_ref, w_ref, dinv_ref, o_ref):
    acc = jnp.dot(hs_ref[0], w_ref[0, 0], precision=HIGH, preferred_element_type=F32)
    acc = acc + jnp.dot(hs_ref[1], w_ref[0, 1], precision=HIGH,
                        preferred_element_type=F32)
    o_ref[...] = (dinv_ref[...] * acc)[None]


def _gcn_pre2(hs2, wqq, dinv):
    return pl.pallas_call(
        _gcn_pre2_body,
        grid=(2, 8),
        in_specs=[
            pl.BlockSpec((2, _RB, 128), lambda q, i: (0, i, 0)),
            pl.BlockSpec((1, 2, 128, 128), lambda q, i: (q, 0, 0, 0)),
            pl.BlockSpec((_RB, 1), lambda q, i: (i, 0)),
        ],
        out_specs=pl.BlockSpec((1, _RB, 128), lambda q, i: (q, i, 0)),
        out_shape=jax.ShapeDtypeStruct((2, NS, 128), F32),
    )(hs2, wqq, dinv)


def _gcn_stat_body(agg_ref, y_ref, dinv_ref, b_ref, u_ref, ps_ref, pss_ref):
    t = dinv_ref[...] * (agg_ref[0] + y_ref[0]) + b_ref[0]
    u_ref[...] = t[None]
    ps_ref[...] = jnp.sum(t, axis=0, keepdims=True)[None, None]
    pss_ref[...] = jnp.sum(t * t, axis=0, keepdims=True)[None, None]


def _gcn_scale_body(ps_ref, pss_ref, g_ref, bt_ref, sc_ref, sh_ref):
    m = jnp.sum(ps_ref[...], axis=1) / NS
    v = jnp.sum(pss_ref[...], axis=1) / NS - m * m
    sc = g_ref[...] / jnp.sqrt(v + 1e-5)
    sc_ref[...] = sc
    sh_ref[...] = bt_ref[...] - m * sc


def _gcn_app_body(u_ref, sc_ref, sh_ref, o_ref):
    o_ref[...] = jnp.maximum(u_ref[...] * sc_ref[...] + sh_ref[...], 0.0)


def _gcn_post(agg2, y2, dinv, lp):
    vec = pl.BlockSpec((1, 1, 128), lambda q, i: (q, 0, 0))
    big = pl.BlockSpec((1, _RB, 128), lambda q, i: (q, i, 0))
    part = pl.BlockSpec((1, 1, 1, 128), lambda q, i: (q, i, 0, 0))
    u, ps, pss = pl.pallas_call(
        _gcn_stat_body,
        grid=(2, 8),
        in_specs=[
            big, big,
            pl.BlockSpec((_RB, 1), lambda q, i: (i, 0)),
            vec,
        ],
        out_specs=[big, part, part],
        out_shape=[
            jax.ShapeDtypeStruct((2, NS, 128), F32),
            jax.ShapeDtypeStruct((2, 8, 1, 128), F32),
            jax.ShapeDtypeStruct((2, 8, 1, 128), F32),
        ],
    )(agg2, y2, dinv, jnp.reshape(lp["b"], (2, 1, 128)))
    sc, sh = pl.pallas_call(
        _gcn_scale_body,
        out_shape=[
            jax.ShapeDtypeStruct((2, 1, 128), F32),
            jax.ShapeDtypeStruct((2, 1, 128), F32),
        ],
    )(ps, pss, jnp.reshape(lp["g"], (2, 1, 128)),
      jnp.reshape(lp["bt"], (2, 1, 128)))
    return pl.pallas_call(
        _gcn_app_body,
        grid=(2, 8),
        in_specs=[big, vec, vec],
        out_specs=big,
        out_shape=jax.ShapeDtypeStruct((2, NS, 128), F32),
    )(u, sc, sh)


def _svc_pool_body(hs_ref, w_ref, b_ref, o_ref):
    i = pl.program_id(0)
    hl = jnp.dot(hs_ref[0], w_ref[0], precision=HIGH, preferred_element_type=F32)
    hl = hl + jnp.dot(hs_ref[1], w_ref[1], precision=HIGH,
                      preferred_element_type=F32)
    part = jnp.sum(jnp.reshape(hl, (_RB // OUT, OUT, H)), axis=0)

    @pl.when(i == 0)
    def _():
        o_ref[...] = part

    @pl.when(i > 0)
    def _():
        o_ref[...] = o_ref[...] + part

    @pl.when(i == 7)
    def _():
        o_ref[...] = o_ref[...] / B + b_ref[...]


def _svc_pool(hs2, svc_w2, b):
    return pl.pallas_call(
        _svc_pool_body,
        grid=(8,),
        in_specs=[
            pl.BlockSpec((2, _RB, 128), lambda i: (0, i, 0)),
            pl.BlockSpec((2, 128, H), lambda i: (0, 0, 0)),
            pl.BlockSpec((1, H), lambda i: (0, 0)),
        ],
        out_specs=pl.BlockSpec((OUT, H), lambda i: (0, 0)),
        out_shape=jax.ShapeDtypeStruct((OUT, H), F32),
    )(hs2, svc_w2, b)


_NPB = N // 10


def _node_pool_body(h_ref, w_ref, b_ref, batch_ref, o_ref, cnt_ref):
    i = pl.program_id(0)
    hl = jnp.dot(h_ref[...], w_ref[...], precision=HIGH,
                 preferred_element_type=F32) + b_ref[...]
    bt = batch_ref[0]
    oh = (lax.broadcasted_iota(jnp.int32, (B, _NPB), 0) == bt).astype(F32)
    s = jnp.dot(oh, hl, precision=HIGH, preferred_element_type=F32)
    cnt = jnp.sum(oh, axis=1, keepdims=True) + jnp.zeros((B, H), F32)

    @pl.when(i == 0)
    def _():
        o_ref[...] = s
        cnt_ref[...] = cnt

    @pl.when(i > 0)
    def _():
        o_ref[...] = o_ref[...] + s
        cnt_ref[...] = cnt_ref[...] + cnt

    @pl.when(i == 9)
    def _():
        o_ref[...] = o_ref[...] / jnp.maximum(cnt_ref[...], 1.0)


def _node_pool(h, w, b, batch3):
    return pl.pallas_call(
        _node_pool_body,
        grid=(10,),
        in_specs=[
            pl.BlockSpec((_NPB, H), lambda i: (i, 0)),
            pl.BlockSpec((H, H), lambda i: (0, 0)),
            pl.BlockSpec((1, H), lambda i: (0, 0)),
            pl.BlockSpec((1, 1, _NPB), lambda i: (i, 0, 0)),
        ],
        out_specs=pl.BlockSpec((B, H), lambda i: (0, 0)),
        out_shape=jax.ShapeDtypeStruct((B, H), F32),
        scratch_shapes=[pltpu.VMEM((B, H), F32)],
    )(h, w, b, batch3)


def _final_body(xg_ref, xs_ref, o_ref):
    logits = lax.dot_general(xg_ref[...], xs_ref[...], (((1,), (1,)), ((), ())),
                             precision=HIGH, preferred_element_type=F32)
    o_ref[...] = jax.nn.sigmoid(logits)


def _pad_rows(w, rows):
    return jnp.concatenate([w, jnp.zeros((rows - w.shape[0], w.shape[1]), w.dtype)], 0)


def kernel(x, x_service, edge_attr_service, params, edge_index, edge_index_service, batch):
    row = edge_index[0].astype(jnp.int32)
    col = edge_index[1].astype(jnp.int32)
    row3 = jnp.reshape(row, (_NC * _NSUB, _GIN_NCH, _GIN_K))
    col3 = jnp.reshape(col, (_NC * _NSUB, _GIN_NCH, _GIN_K))
    rows_s = edge_index_service[0].astype(jnp.int32)
    cols_s = edge_index_service[1].astype(jnp.int32)
    ecat = jnp.concatenate(
        [jnp.reshape(rows_s, (ES // _GCN_K, _GCN_K)),
         jnp.reshape(cols_s, (ES // _GCN_K, _GCN_K)),
         jax.lax.bitcast_convert_type(
             jnp.reshape(edge_attr_service, (ES // _GCN_K, _GCN_K)), jnp.int32)],
        axis=0)
    emb_n = _pad_rows(params["node_emb"], 128)
    emb_s = _pad_rows(params["svc_emb"], 128)
    gin_w1 = [_pad_rows(params["gin"][0]["W1"], 144),
              params["gin"][1]["W1"], params["gin"][2]["W1"]]
    gcn_w1 = jnp.transpose(
        jnp.reshape(_pad_rows(params["gcn"][0]["W"], 144), (144, 2, 128)), (1, 0, 2))
    gcn_w2 = jnp.transpose(
        jnp.reshape(params["gcn"][1]["W"], (2, 128, 2, 128)), (2, 0, 1, 3))
    svc_w2 = jnp.reshape(params["svcLin_W"], (2, 128, H))
    batch3 = jnp.reshape(batch.astype(jnp.int32), (10, 1, _NPB))

    h = _embed(x, emb_n, 7)
    lp = params["gin"][0]
    pa = _gin_agg_kernel(h[:, 0:128], row3, col3)
    pc = _gin_agg_kernel(h[:, 16:144], row3, col3)
    agg1 = _gin_comb(pa, pc)
    h = pl.pallas_call(
        _gin_dense1_body,
        out_shape=jax.ShapeDtypeStruct((N, H), F32),
    )(h, agg1, *_gin_params(lp, gin_w1[0]))
    for i in (1, 2):
        p = _gin_agg_kernel(h, row3, col3)
        h = pl.pallas_call(
            _gin_dense_body,
            out_shape=jax.ShapeDtypeStruct((N, H), F32),
        )(h, p, *_gin_params(params["gin"][i], gin_w1[i]))

    hs = _embed(x_service, emb_s, 5)
    degp = _deg_kernel(cols_s, edge_attr_service)
    deg = jnp.sum(degp, axis=0) + 1.0
    dinv = jnp.reshape(1.0 / jnp.sqrt(deg), (NS, 1))
    y2 = _gcn_pre1(hs, gcn_w1, dinv)
    agg2 = _gcn_agg_kernel(jnp.reshape(y2, (2 * NS, 128)), ecat)
    hs2 = _gcn_post(jnp.reshape(agg2, (2, NS, 128)), y2, dinv, params["gcn"][0])
    y2 = _gcn_pre2(hs2, gcn_w2, dinv)
    agg2 = _gcn_agg_kernel(jnp.reshape(y2, (2 * NS, 128)), ecat)
    hs2 = _gcn_post(jnp.reshape(agg2, (2, NS, 128)), y2, dinv, params["gcn"][1])

    xs = _svc_pool(hs2, svc_w2, jnp.reshape(params["svcLin_b"], (1, -1)))
    xg = _node_pool(h, params["nodeLin_W"],
                    jnp.reshape(params["nodeLin_b"], (1, -1)), batch3)
    return pl.pallas_call(
        _final_body,
        out_shape=jax.ShapeDtypeStruct((B, OUT), F32),
    )(xg, xs)
